# baseline (speedup 1.0000x reference)
import functools

import jax
import jax.numpy as jnp
from jax import lax
from jax.experimental import pallas as pl
from jax.experimental.pallas import tpu as pltpu
from jax.experimental.pallas import tpu_sc as plsc
from jax.experimental.layout import Layout, with_layout_constraint

BATCH = 4096
SEQ = 200
EMBED = 64
NUM_WORKERS = 32
BPW = BATCH // NUM_WORKERS
NBUF = 3

_MESH = plsc.VectorSubcoreMesh(
    core_axis_name="c", subcore_axis_name="s", num_cores=2, num_subcores=16
)


@functools.partial(
    pl.kernel,
    out_type=jax.ShapeDtypeStruct((SEQ, EMBED // 8, NUM_WORKERS, 8, 128),
                                  jnp.float32),
    mesh=_MESH,
    compiler_params=pltpu.CompilerParams(use_tc_tiling_on_sc=False,
                                         needs_layout_passes=False),
    scratch_types=[
        pltpu.VMEM((SEQ, BPW), jnp.int32),
        pltpu.VMEM((SEQ, BPW), jnp.int32),
    ]
    + [pltpu.VMEM((BPW, EMBED), jnp.float32) for _ in range(NBUF)]
    + [pltpu.VMEM((EMBED // 8, 8, 128), jnp.float32) for _ in range(NBUF)]
    + [pltpu.SemaphoreType.DMA] * (3 * NBUF),
)
def _embed_sum(notes_hbm, times_hbm, tok_hbm, pos_hbm, out_hbm,
               idx_n, idx_t, *bufs_and_sems):
    bufs = bufs_and_sems[:NBUF]
    tbufs = bufs_and_sems[NBUF:2 * NBUF]
    sem_a = bufs_and_sems[2 * NBUF:3 * NBUF]
    sem_b = bufs_and_sems[3 * NBUF:4 * NBUF]
    sem_c = bufs_and_sems[4 * NBUF:5 * NBUF]

    w = lax.axis_index("s") * 2 + lax.axis_index("c")
    pltpu.sync_copy(notes_hbm.at[pl.ds(0, SEQ), w], idx_n)
    pltpu.sync_copy(times_hbm.at[pl.ds(0, SEQ), w], idx_t)

    lanes = lax.iota(jnp.int32, 16)

    def transpose_chunk(b):
        def body(m, carry):
            for q in range(8):
                cols = jnp.full((16,), m * 8 + q, jnp.int32)
                for k in range(8):
                    v = plsc.load_gather(bufs[b], [k * 16 + lanes, cols])
                    tbufs[b][m, q, pl.ds(k * 16, 16)] = v
            return carry

        lax.fori_loop(0, EMBED // 8, body, 0)

    def step(g):
        for b in range(NBUF):
            c = g + b
            b1 = (b - 1) % NBUF
            b2 = (b - 2) % NBUF

            @pl.when(c < SEQ)
            def _():
                pltpu.async_copy(tok_hbm.at[idx_n.at[c]], bufs[b], sem_a[b])

            c1 = c - 1
            @pl.when(jnp.logical_and(c1 >= 0, c1 < SEQ))
            def _():
                pltpu.make_async_copy(
                    tok_hbm.at[idx_n.at[c1]], bufs[b1], sem_a[b1]
                ).wait()
                pltpu.async_copy(
                    pos_hbm.at[idx_t.at[c1]], bufs[b1], sem_b[b1], add=True
                )

            c2 = c - 2
            @pl.when(jnp.logical_and(c2 >= 0, c2 < SEQ))
            def _():
                pltpu.make_async_copy(
                    pos_hbm.at[idx_t.at[c2]], bufs[b2], sem_b[b2]
                ).wait()

                @pl.when(c2 >= NBUF)
                def _():
                    pltpu.make_async_copy(
                        tbufs[b2], out_hbm.at[0, pl.ds(0, EMBED // 8), w],
                        sem_c[b2]
                    ).wait()

                transpose_chunk(b2)
                pltpu.async_copy(
                    tbufs[b2], out_hbm.at[c2, pl.ds(0, EMBED // 8), w],
                    sem_c[b2]
                )

    pl.loop(0, SEQ + 3 * NBUF, step=NBUF)(step)

    for b in range(NBUF):
        c_last = SEQ - NBUF + b
        pltpu.make_async_copy(
            tbufs[b % NBUF], out_hbm.at[0, pl.ds(0, EMBED // 8), w],
            sem_c[c_last % NBUF]
        ).wait()


def kernel(x, token_emb, pos_emb):
    notes_t = jnp.transpose(x[:, 0, :].astype(jnp.int32)).reshape(
        SEQ, NUM_WORKERS, BPW)
    times_t = jnp.transpose(x[:, 1, :].astype(jnp.int32)).reshape(
        SEQ, NUM_WORKERS, BPW)
    out5 = _embed_sum(notes_t, times_t, token_emb, pos_emb)
    t = jnp.transpose(out5, (2, 4, 0, 1, 3))
    r = t.reshape(BATCH, SEQ, EMBED)
    return with_layout_constraint(r, Layout(major_to_minor=(1, 2, 0)))


# speedup vs baseline: 1.6670x; 1.6670x over previous
import functools

import jax
import jax.numpy as jnp
from jax import lax
from jax.experimental import pallas as pl
from jax.experimental.pallas import tpu as pltpu
from jax.experimental.pallas import tpu_sc as plsc
from jax.experimental.layout import Layout, with_layout_constraint

BATCH = 4096
SEQ = 200
EMBED = 64
NUM_WORKERS = 32
BPW = BATCH // NUM_WORKERS
NBUF = 3
PITCH = 129

_MESH = plsc.VectorSubcoreMesh(
    core_axis_name="c", subcore_axis_name="s", num_cores=2, num_subcores=16
)


@functools.partial(
    pl.kernel,
    out_type=jax.ShapeDtypeStruct((SEQ, EMBED // 8, NUM_WORKERS, 8, 128),
                                  jnp.float32),
    mesh=_MESH,
    compiler_params=pltpu.CompilerParams(use_tc_tiling_on_sc=False,
                                         needs_layout_passes=False),
    scratch_types=[
        pltpu.VMEM((SEQ, BPW), jnp.int32),
        pltpu.VMEM((SEQ, BPW), jnp.int32),
    ]
    + [pltpu.VMEM((BPW, EMBED), jnp.float32) for _ in range(NBUF)]
    + [pltpu.VMEM((EMBED // 8, 8, 128), jnp.float32) for _ in range(NBUF)]
    + [pltpu.VMEM((EMBED * PITCH,), jnp.float32) for _ in range(NBUF)]
    + [pltpu.SemaphoreType.DMA] * (3 * NBUF),
)
def _embed_sum(notes_hbm, times_hbm, tok_hbm, pos_hbm, out_hbm,
               idx_n, idx_t, *bufs_and_sems):
    bufs = bufs_and_sems[:NBUF]
    tbufs = bufs_and_sems[NBUF:2 * NBUF]
    pbufs = bufs_and_sems[2 * NBUF:3 * NBUF]
    sem_a = bufs_and_sems[3 * NBUF:4 * NBUF]
    sem_b = bufs_and_sems[4 * NBUF:5 * NBUF]
    sem_c = bufs_and_sems[5 * NBUF:6 * NBUF]

    w = lax.axis_index("s") * 2 + lax.axis_index("c")
    pltpu.sync_copy(notes_hbm.at[pl.ds(0, SEQ), w], idx_n)
    pltpu.sync_copy(times_hbm.at[pl.ds(0, SEQ), w], idx_t)

    lanes = lax.iota(jnp.int32, 16)

    offs = [(16 * m + lanes) * PITCH for m in range(EMBED // 16)]

    def transpose_chunk(b):
        def scat(j0, carry):
            for dj in range(4):
                j = j0 * 4 + dj
                for m in range(EMBED // 16):
                    v = bufs[b][j, pl.ds(16 * m, 16)]
                    plsc.store_scatter(pbufs[b], [offs[m] + j], v)
            return carry

        lax.fori_loop(0, BPW // 4, scat, 0)

        def repack(m, carry):
            for q in range(8):
                e = m * 8 + q
                for k in range(8):
                    tbufs[b][m, q, pl.ds(16 * k, 16)] = (
                        pbufs[b][pl.ds(e * PITCH + 16 * k, 16)])
            return carry

        lax.fori_loop(0, EMBED // 8, repack, 0)

    def step(g):
        for b in range(NBUF):
            c = g + b
            b1 = (b - 1) % NBUF
            b2 = (b - 2) % NBUF

            @pl.when(c < SEQ)
            def _():
                pltpu.async_copy(tok_hbm.at[idx_n.at[c]], bufs[b], sem_a[b])

            c1 = c - 1
            @pl.when(jnp.logical_and(c1 >= 0, c1 < SEQ))
            def _():
                pltpu.make_async_copy(
                    tok_hbm.at[idx_n.at[c1]], bufs[b1], sem_a[b1]
                ).wait()
                pltpu.async_copy(
                    pos_hbm.at[idx_t.at[c1]], bufs[b1], sem_b[b1], add=True
                )

            c2 = c - 2
            @pl.when(jnp.logical_and(c2 >= 0, c2 < SEQ))
            def _():
                pltpu.make_async_copy(
                    pos_hbm.at[idx_t.at[c2]], bufs[b2], sem_b[b2]
                ).wait()

                @pl.when(c2 >= NBUF)
                def _():
                    pltpu.make_async_copy(
                        tbufs[b2], out_hbm.at[0, pl.ds(0, EMBED // 8), w],
                        sem_c[b2]
                    ).wait()

                transpose_chunk(b2)
                pltpu.async_copy(
                    tbufs[b2], out_hbm.at[c2, pl.ds(0, EMBED // 8), w],
                    sem_c[b2]
                )

    pl.loop(0, SEQ + 3 * NBUF, step=NBUF)(step)

    for b in range(NBUF):
        c_last = SEQ - NBUF + b
        pltpu.make_async_copy(
            tbufs[b % NBUF], out_hbm.at[0, pl.ds(0, EMBED // 8), w],
            sem_c[c_last % NBUF]
        ).wait()


def kernel(x, token_emb, pos_emb):
    notes_t = jnp.transpose(x[:, 0, :].astype(jnp.int32)).reshape(
        SEQ, NUM_WORKERS, BPW)
    times_t = jnp.transpose(x[:, 1, :].astype(jnp.int32)).reshape(
        SEQ, NUM_WORKERS, BPW)
    out5 = _embed_sum(notes_t, times_t, token_emb, pos_emb)
    t = jnp.transpose(out5, (2, 4, 0, 1, 3))
    r = t.reshape(BATCH, SEQ, EMBED)
    return with_layout_constraint(r, Layout(major_to_minor=(1, 2, 0)))
